# baseline (speedup 1.0000x reference)
import functools

import jax
import jax.numpy as jnp
from jax import lax
from jax.experimental import pallas as pl
from jax.experimental.pallas import tpu as pltpu
from jax.experimental.pallas import tpu_sc as plsc

ED = 32
NCTX = 5
B = 16384
NC = 2
NS = 16
NW = NC * NS
BPW = B // NW
PPW = BPW * NCTX
CHUNK = 128


def _sc_body(tgt_hbm, ctx_hbm, ttab_hbm, ctab_hbm, out_hbm,
             tidx_v, cidx_v, trows_v, crows_v, out_v, sem):
    wid = lax.axis_index("s") * NC + lax.axis_index("c")
    bbase = wid * BPW
    pbase = wid * PPW

    pltpu.sync_copy(tgt_hbm.at[pl.ds(bbase, BPW)], tidx_v)
    pltpu.sync_copy(ctx_hbm.at[pl.ds(pbase, PPW)], cidx_v)

    copies = []
    for j in range(BPW // CHUNK):
        copies.append(pltpu.make_async_copy(
            ttab_hbm.at[tidx_v.at[pl.ds(j * CHUNK, CHUNK)]],
            trows_v.at[pl.ds(j * CHUNK, CHUNK)],
            sem))
    for j in range(PPW // CHUNK):
        copies.append(pltpu.make_async_copy(
            ctab_hbm.at[cidx_v.at[pl.ds(j * CHUNK, CHUNK)]],
            crows_v.at[pl.ds(j * CHUNK, CHUNK)],
            sem))
    for cp in copies:
        cp.start()
    for cp in copies:
        cp.wait()

    lane = lax.iota(jnp.int32, 16)

    def body(i, carry):
        b0 = i * 16
        accs = [jnp.zeros((16,), jnp.float32) for _ in range(NCTX)]
        for bb in range(16):
            b = b0 + bb
            we0, we1 = plsc.unpack(
                trows_v[b, :], format=plsc.PackFormat.INTERLEAVED)
            for c in range(NCTX):
                lp = bb * NCTX + c
                p = b * NCTX + c
                ce0, ce1 = plsc.unpack(
                    crows_v[p, :], format=plsc.PackFormat.INTERLEAVED)
                v = ce0 * we0 + ce1 * we1
                s = jnp.sum(v)
                g, ln = divmod(lp, 16)
                accs[g] = jnp.where(lane == ln, s, accs[g])
        for g in range(NCTX):
            out_v[pl.ds(b0 * NCTX + g * 16, 16)] = accs[g]
        return carry

    lax.fori_loop(0, BPW // 16, body, 0)
    pltpu.sync_copy(out_v, out_hbm.at[pl.ds(pbase, PPW)])


def kernel(target, context, target_table, context_table):
    tgt = target.reshape(B)
    ctx = context.reshape(B * NCTX)
    ttab = target_table.astype(jnp.bfloat16)
    ctab = context_table.astype(jnp.bfloat16)
    mesh = plsc.VectorSubcoreMesh(core_axis_name="c", subcore_axis_name="s")
    run = functools.partial(
        pl.kernel,
        mesh=mesh,
        out_type=jax.ShapeDtypeStruct((B * NCTX,), jnp.float32),
        scratch_types=[
            pltpu.VMEM((BPW,), jnp.int32),
            pltpu.VMEM((PPW,), jnp.int32),
            pltpu.VMEM((BPW, ED), jnp.bfloat16),
            pltpu.VMEM((PPW, ED), jnp.bfloat16),
            pltpu.VMEM((PPW,), jnp.float32),
            pltpu.SemaphoreType.DMA,
        ],
        compiler_params=pltpu.CompilerParams(
            needs_layout_passes=False, use_tc_tiling_on_sc=False),
    )(_sc_body)
    out = run(tgt, ctx, ttab, ctab)
    return out.reshape(B, NCTX)


# speedup vs baseline: 1.1719x; 1.1719x over previous
import functools

import jax
import jax.numpy as jnp
from jax import lax
from jax.experimental import pallas as pl
from jax.experimental.pallas import tpu as pltpu
from jax.experimental.pallas import tpu_sc as plsc

ED = 32
NCTX = 5
B = 16384
NC = 2
NS = 16
NW = NC * NS
BPW = B // NW
PPW = BPW * NCTX
CHUNK = 128


def _sc_body(tgt_hbm, ctx_hbm, ttab_hbm, ctab_hbm, out_hbm,
             tidx_v, cidx_v, trows_v, crows_v, out_v, sem):
    wid = lax.axis_index("s") * NC + lax.axis_index("c")
    bbase = wid * BPW
    pbase = wid * PPW

    pltpu.sync_copy(tgt_hbm.at[pl.ds(bbase, BPW)], tidx_v)
    pltpu.sync_copy(ctx_hbm.at[pl.ds(pbase, PPW)], cidx_v)

    copies = []
    for j in range(BPW // CHUNK):
        copies.append(pltpu.make_async_copy(
            ttab_hbm.at[tidx_v.at[pl.ds(j * CHUNK, CHUNK)]],
            trows_v.at[pl.ds(j * CHUNK, CHUNK)],
            sem))
    for j in range(PPW // CHUNK):
        copies.append(pltpu.make_async_copy(
            ctab_hbm.at[cidx_v.at[pl.ds(j * CHUNK, CHUNK)]],
            crows_v.at[pl.ds(j * CHUNK, CHUNK)],
            sem))
    for cp in copies:
        cp.start()
    for cp in copies:
        cp.wait()

    lane = lax.iota(jnp.int32, 16)

    def body(i, carry):
        b0 = i * 16
        accs = [jnp.zeros((16,), jnp.float32) for _ in range(NCTX)]
        for bb in range(16):
            b = b0 + bb
            we0 = trows_v[b, pl.ds(0, 16)]
            we1 = trows_v[b, pl.ds(16, 16)]
            for c in range(NCTX):
                lp = bb * NCTX + c
                p = b * NCTX + c
                v = (crows_v[p, pl.ds(0, 16)] * we0
                     + crows_v[p, pl.ds(16, 16)] * we1)
                s = jnp.sum(v)
                g, ln = divmod(lp, 16)
                accs[g] = jnp.where(lane == ln, s, accs[g])
        for g in range(NCTX):
            out_v[pl.ds(b0 * NCTX + g * 16, 16)] = accs[g]
        return carry

    lax.fori_loop(0, BPW // 16, body, 0)
    pltpu.sync_copy(out_v, out_hbm.at[pl.ds(pbase, PPW)])


def kernel(target, context, target_table, context_table):
    tgt = target.reshape(B)
    ctx = context.reshape(B * NCTX)
    mesh = plsc.VectorSubcoreMesh(core_axis_name="c", subcore_axis_name="s")
    run = functools.partial(
        pl.kernel,
        mesh=mesh,
        out_type=jax.ShapeDtypeStruct((B * NCTX,), jnp.float32),
        scratch_types=[
            pltpu.VMEM((BPW,), jnp.int32),
            pltpu.VMEM((PPW,), jnp.int32),
            pltpu.VMEM((BPW, ED), jnp.float32),
            pltpu.VMEM((PPW, ED), jnp.float32),
            pltpu.VMEM((PPW,), jnp.float32),
            pltpu.SemaphoreType.DMA,
        ],
        compiler_params=pltpu.CompilerParams(
            needs_layout_passes=False, use_tc_tiling_on_sc=False),
    )(_sc_body)
    out = run(tgt, ctx, target_table, context_table)
    return out.reshape(B, NCTX)
